# baseline (speedup 1.0000x reference)
import functools

import jax
import jax.numpy as jnp
from jax import lax
from jax.experimental import pallas as pl
from jax.experimental.pallas import tpu as pltpu
from jax.experimental.pallas import tpu_sc as plsc

_NC, _NS, _L = 2, 16, 16
_NW = _NC * _NS
_H = 64
_IDXW = 80
_EPS = 1e-8


_GATHER_DN = lax.GatherDimensionNumbers(
    offset_dims=(), collapsed_slice_dims=(0,), start_index_map=(0,))


def _lanes_perm(v, idx):
    return lax.gather(v, idx[:, None], _GATHER_DN, slice_sizes=(1,),
                      mode=lax.GatherScatterMode.PROMISE_IN_BOUNDS)


def _sum16(v, lane):
    for sh in (8, 4, 2, 1):
        v = v + _lanes_perm(v, lane ^ sh)
    return v


def _rsqrt16(v):
    i = lax.bitcast_convert_type(v, jnp.int32)
    y = lax.bitcast_convert_type(jnp.int32(0x5F3759DF) - (i >> 1), jnp.float32)
    h = v * jnp.float32(-0.5)
    for _ in range(3):
        y = y * (jnp.float32(1.5) + h * y * y)
    return y


def _tec_body(C, T, x1d, W_words, W_pos, scale_h, bias_h, out,
              idx_v, rows_v, pos2_v, s_v, b_v, sem):
    S = C // 2
    tpw = T // _NW
    nchunk = tpw // C
    wid = lax.axis_index("s") * _NC + lax.axis_index("c")

    pltpu.sync_copy(W_pos.at[pl.ds(0, S)], pos2_v.at[pl.ds(0, S)])
    pltpu.sync_copy(W_pos.at[pl.ds(0, S)], pos2_v.at[pl.ds(S, S)])
    pltpu.sync_copy(scale_h, s_v)
    pltpu.sync_copy(bias_h, b_v)

    sregs = tuple(s_v[pl.ds(_L * j, _L)] for j in range(_H // _L))
    bregs = tuple(b_v[pl.ds(_L * j, _L)] for j in range(_H // _L))

    lane = lax.iota(jnp.int32, _L)

    def token(t, carry):
        sr, br = carry
        e = []
        for j in range(_H // _L):
            w = rows_v[t, pl.ds(_L * j, _L)]
            p = pos2_v[t, pl.ds(_L * j, _L)]
            e.append(w + p)
        sv = (e[0] + e[1]) + (e[2] + e[3])
        sq = (e[0] * e[0] + e[1] * e[1]) + (e[2] * e[2] + e[3] * e[3])
        tot = _sum16(sv, lane)
        tot2 = _sum16(sq, lane)
        mean = tot * jnp.float32(1.0 / _H)
        var = tot2 * jnp.float32(1.0 / _H) - mean * mean + jnp.float32(_EPS)
        r = _rsqrt16(var)
        for j in range(_H // _L):
            o = (e[j] - mean) * r * sr[j] + br[j]
            rows_v[t, pl.ds(_L * j, _L)] = o
        return carry

    def tok_loop(i, carry):
        t0 = i * 4
        for u in range(4):
            carry = token(t0 + u, carry)
        return carry

    def chunk(g, carry):
        tok0 = wid * tpw + g * C
        pltpu.sync_copy(x1d.at[pl.ds(tok0, C)], idx_v)
        copies = [
            pltpu.async_copy(W_words.at[idx_v.at[pl.ds(_IDXW * j, _IDXW)]],
                             rows_v.at[pl.ds(_IDXW * j, _IDXW)], sem)
            for j in range(C // _IDXW)
        ]
        for c in copies:
            c.wait()
        carry = lax.fori_loop(0, C // 4, tok_loop, carry)
        pltpu.sync_copy(rows_v, out.at[pl.ds(tok0, C)])
        return carry

    lax.fori_loop(0, nchunk, chunk, (sregs, bregs))


@functools.partial(jax.jit, static_argnames=())
def kernel(x, W_words, W_pos, ln_scale, ln_bias):
    B, S = x.shape
    H = W_words.shape[1]
    T = B * S
    C = 2 * S
    assert H == _H and T % (_NW * C) == 0 and C % _IDXW == 0 and S % 8 == 0

    x1d = x.astype(jnp.int32).reshape(T)
    mesh = plsc.VectorSubcoreMesh(core_axis_name="c", subcore_axis_name="s",
                                  num_cores=_NC, num_subcores=_NS)
    run = pl.kernel(
        functools.partial(_tec_body, C, T),
        out_type=jax.ShapeDtypeStruct((T, H), jnp.float32),
        mesh=mesh,
        compiler_params=pltpu.CompilerParams(use_tc_tiling_on_sc=False),
        scratch_types=[
            pltpu.VMEM((C,), jnp.int32),
            pltpu.VMEM((C, H), jnp.float32),
            pltpu.VMEM((C, H), jnp.float32),
            pltpu.VMEM((H,), jnp.float32),
            pltpu.VMEM((H,), jnp.float32),
            pltpu.SemaphoreType.DMA,
        ],
    )
    out = run(x1d, W_words, W_pos, ln_scale, ln_bias)
    return out.reshape(B, S, H)


# speedup vs baseline: 1.0217x; 1.0217x over previous
import functools

import jax
import jax.numpy as jnp
from jax import lax
from jax.experimental import pallas as pl
from jax.experimental.pallas import tpu as pltpu
from jax.experimental.pallas import tpu_sc as plsc

_NC, _NS, _L = 2, 16, 16
_NW = _NC * _NS
_H = 64
_IDXW = 80
_EPS = 1e-8


_GATHER_DN = lax.GatherDimensionNumbers(
    offset_dims=(), collapsed_slice_dims=(0,), start_index_map=(0,))


def _lanes_perm(v, idx):
    return lax.gather(v, idx[:, None], _GATHER_DN, slice_sizes=(1,),
                      mode=lax.GatherScatterMode.PROMISE_IN_BOUNDS)


def _sum16(v, lane):
    for sh in (8, 4, 2, 1):
        v = v + _lanes_perm(v, lane ^ sh)
    return v


def _rsqrt16(v):
    i = lax.bitcast_convert_type(v, jnp.int32)
    y = lax.bitcast_convert_type(jnp.int32(0x5F3759DF) - (i >> 1), jnp.float32)
    h = v * jnp.float32(-0.5)
    for _ in range(3):
        y = y * (jnp.float32(1.5) + h * y * y)
    return y


def _tec_body(C, T, x1d, W_words, W_pos, scale_h, bias_h, out,
              idx_v, rows_v, pos2_v, s_v, b_v, sem):
    S = C // 2
    tpw = T // _NW
    nchunk = tpw // C
    wid = lax.axis_index("s") * _NC + lax.axis_index("c")

    pltpu.sync_copy(W_pos.at[pl.ds(0, S)], pos2_v.at[pl.ds(0, S)])
    pltpu.sync_copy(W_pos.at[pl.ds(0, S)], pos2_v.at[pl.ds(S, S)])
    pltpu.sync_copy(scale_h, s_v)
    pltpu.sync_copy(bias_h, b_v)

    sregs = tuple(s_v[pl.ds(_L * j, _L)] for j in range(_H // _L))
    bregs = tuple(b_v[pl.ds(_L * j, _L)] for j in range(_H // _L))

    lane = lax.iota(jnp.int32, _L)

    def token(t):
        e = []
        for j in range(_H // _L):
            w = rows_v[t, pl.ds(_L * j, _L)]
            p = pos2_v[t, pl.ds(_L * j, _L)]
            e.append(w + p)
        sv = (e[0] + e[1]) + (e[2] + e[3])
        sq = (e[0] * e[0] + e[1] * e[1]) + (e[2] * e[2] + e[3] * e[3])
        tot = _sum16(sv, lane)
        tot2 = _sum16(sq, lane)
        mean = tot * jnp.float32(1.0 / _H)
        var = tot2 * jnp.float32(1.0 / _H) - mean * mean + jnp.float32(_EPS)
        r = _rsqrt16(var)
        for j in range(_H // _L):
            rs = r * sregs[j]
            rows_v[t, pl.ds(_L * j, _L)] = (e[j] - mean) * rs + bregs[j]

    def chunk(g, carry):
        tok0 = wid * tpw + g * C
        pltpu.sync_copy(x1d.at[pl.ds(tok0, C)], idx_v)
        copies = [
            pltpu.async_copy(W_words.at[idx_v.at[pl.ds(_IDXW * j, _IDXW)]],
                             rows_v.at[pl.ds(_IDXW * j, _IDXW)], sem)
            for j in range(C // _IDXW)
        ]
        for c in copies:
            c.wait()
        plsc.parallel_loop(0, C, step=1, unroll=8)(token)
        pltpu.sync_copy(rows_v, out.at[pl.ds(tok0, C)])
        return carry

    lax.fori_loop(0, nchunk, chunk, 0)


@functools.partial(jax.jit, static_argnames=())
def kernel(x, W_words, W_pos, ln_scale, ln_bias):
    B, S = x.shape
    H = W_words.shape[1]
    T = B * S
    C = 2 * S
    assert H == _H and T % (_NW * C) == 0 and C % _IDXW == 0 and S % 8 == 0

    x1d = x.astype(jnp.int32).reshape(T)
    mesh = plsc.VectorSubcoreMesh(core_axis_name="c", subcore_axis_name="s",
                                  num_cores=_NC, num_subcores=_NS)
    run = pl.kernel(
        functools.partial(_tec_body, C, T),
        out_type=jax.ShapeDtypeStruct((T, H), jnp.float32),
        mesh=mesh,
        compiler_params=pltpu.CompilerParams(use_tc_tiling_on_sc=False),
        scratch_types=[
            pltpu.VMEM((C,), jnp.int32),
            pltpu.VMEM((C, H), jnp.float32),
            pltpu.VMEM((C, H), jnp.float32),
            pltpu.VMEM((H,), jnp.float32),
            pltpu.VMEM((H,), jnp.float32),
            pltpu.SemaphoreType.DMA,
        ],
    )
    out = run(x1d, W_words, W_pos, ln_scale, ln_bias)
    return out.reshape(B, S, H)


# speedup vs baseline: 1.1792x; 1.1542x over previous
import functools

import jax
import jax.numpy as jnp
from jax import lax
from jax.experimental import pallas as pl
from jax.experimental.pallas import tpu as pltpu
from jax.experimental.pallas import tpu_sc as plsc

_NC, _NS, _L = 2, 16, 16
_NW = _NC * _NS
_H = 64
_EPS = 1e-8

_GATHER_DN = lax.GatherDimensionNumbers(
    offset_dims=(), collapsed_slice_dims=(0,), start_index_map=(0,))

_SLICES = ((0, 80), (80, 80), (160, 40))


def _lanes_perm(v, idx):
    return lax.gather(v, idx[:, None], _GATHER_DN, slice_sizes=(1,),
                      mode=lax.GatherScatterMode.PROMISE_IN_BOUNDS)


def _sum16(v, lane):
    for sh in (8, 4, 2, 1):
        v = v + _lanes_perm(v, lane ^ sh)
    return v


def _rsqrt16(v):
    i = lax.bitcast_convert_type(v, jnp.int32)
    y = lax.bitcast_convert_type(jnp.int32(0x5F3759DF) - (i >> 1), jnp.float32)
    h = v * jnp.float32(-0.5)
    for _ in range(3):
        y = y * (jnp.float32(1.5) + h * y * y)
    return y


def _tec_body(S, T, x1d, Ww2, Wp2, scale_h, bias_h, out2,
              idx0, idx1, rows0, rows1, ob_v, pos_v, s_v, b_v,
              semi0, semi1, semg0, semg1, semo0, semo1):
    SP = S // 2
    tpw = T // _NW
    nchunk = tpw // S
    npair = nchunk // 2
    wid = lax.axis_index("s") * _NC + lax.axis_index("c")
    idx = (idx0, idx1)
    rows = (rows0, rows1)
    semi = (semi0, semi1)
    semg = (semg0, semg1)
    semo = (semo0, semo1)

    pltpu.sync_copy(Wp2.at[pl.ds(0, 104)], pos_v)
    pltpu.sync_copy(scale_h, s_v)
    pltpu.sync_copy(bias_h, b_v)

    sregs = tuple(s_v[pl.ds(_L * j, _L)] for j in range(_H // _L))
    bregs = tuple(b_v[pl.ds(_L * j, _L)] for j in range(_H // _L))
    lane = lax.iota(jnp.int32, _L)

    def tok0_of(c):
        return pl.multiple_of(wid * tpw + c * S, 8)

    def issue_idx(c, b):
        return pltpu.async_copy(x1d.at[pl.ds(tok0_of(c), S)],
                                idx[b].at[pl.ds(0, S)], semi[b])

    def wait_idx(b):
        pltpu.make_async_copy(x1d.at[pl.ds(tok0_of(0), S)],
                              idx[b].at[pl.ds(0, S)], semi[b]).wait()

    def issue_gather(b):
        for o, n in _SLICES:
            pltpu.async_copy(Ww2.at[idx[b].at[pl.ds(o, n)]],
                             rows[b].at[pl.ds(o, n)], semg[b])

    def wait_gather(b):
        for o, n in _SLICES:
            pltpu.make_async_copy(Ww2.at[idx[b].at[pl.ds(o, n)]],
                                  rows[b].at[pl.ds(o, n)], semg[b]).wait()

    def orow_of(gp):
        return pl.multiple_of((wid * tpw + gp * 2 * S) // 2, 8)

    def issue_scatter(gp, p):
        pltpu.async_copy(ob_v.at[pl.ds(p * S, S)],
                         out2.at[pl.ds(orow_of(gp), S)], semo[p])

    def wait_scatter(p):
        pltpu.make_async_copy(ob_v.at[pl.ds(p * S, S)],
                              out2.at[pl.ds(orow_of(0), S)], semo[p]).wait()

    def ln_one(e):
        sv = (e[0] + e[1]) + (e[2] + e[3])
        sq = (e[0] * e[0] + e[1] * e[1]) + (e[2] * e[2] + e[3] * e[3])
        tot = _sum16(sv, lane)
        tot2 = _sum16(sq, lane)
        mean = tot * jnp.float32(1.0 / _H)
        var = tot2 * jnp.float32(1.0 / _H) - mean * mean + jnp.float32(_EPS)
        r = _rsqrt16(var)
        return [(e[j] - mean) * (r * sregs[j]) + bregs[j]
                for j in range(_H // _L)]

    def compute(b, ob_base):
        def pair(t):
            tp = t >> 1
            for half in (0, 1):
                e = [rows[b][t + half, pl.ds(_L * j, _L)] +
                     pos_v[tp, pl.ds(64 * half + _L * j, _L)]
                     for j in range(_H // _L)]
                o = ln_one(e)
                for j in range(_H // _L):
                    ob_v[ob_base + tp, pl.ds(64 * half + _L * j, _L)] = o[j]
        plsc.parallel_loop(0, S, step=2, unroll=4)(pair)

    issue_idx(0, 0).wait()
    issue_gather(0)
    issue_idx(1, 1)

    def one_pair(gp, p, first, more):
        @pl.when(jnp.logical_not(first))
        def _():
            wait_scatter(p)
        wait_gather(0)
        wait_idx(1)
        issue_gather(1)
        compute(0, p * S)
        @pl.when(more)
        def _():
            issue_idx(2 * gp + 2, 0)
        wait_gather(1)
        @pl.when(more)
        def _():
            wait_idx(0)
            issue_gather(0)
        compute(1, p * S + SP)
        @pl.when(more)
        def _():
            issue_idx(2 * gp + 3, 1)
        issue_scatter(gp, p)

    def two_pairs(i, carry):
        one_pair(2 * i, 0, i < 1, jnp.bool_(True))
        one_pair(2 * i + 1, 1, i < 1, i < npair // 2 - 1)
        return carry

    lax.fori_loop(0, npair // 2, two_pairs, 0)
    wait_scatter(0)
    wait_scatter(1)


@functools.partial(jax.jit, static_argnames=())
def kernel(x, W_words, W_pos, ln_scale, ln_bias):
    B, S = x.shape
    V, H = W_words.shape
    T = B * S
    assert H == _H and T % (_NW * 2 * S) == 0 and S % 8 == 0

    x1d = x.astype(jnp.int32).reshape(T)
    Ww2 = jnp.pad(W_words, ((0, 0), (0, H)))
    Wp2 = W_pos.reshape(W_pos.shape[0] // 2, 2 * H)
    mesh = plsc.VectorSubcoreMesh(core_axis_name="c", subcore_axis_name="s",
                                  num_cores=_NC, num_subcores=_NS)
    run = pl.kernel(
        functools.partial(_tec_body, S, T),
        out_type=jax.ShapeDtypeStruct((T // 2, 2 * H), jnp.float32),
        mesh=mesh,
        scratch_types=[
            pltpu.VMEM((224,), jnp.int32),
            pltpu.VMEM((224,), jnp.int32),
            pltpu.VMEM((S, 2 * H), jnp.float32),
            pltpu.VMEM((S, 2 * H), jnp.float32),
            pltpu.VMEM((2 * S, 2 * H), jnp.float32),
            pltpu.VMEM((104, 2 * H), jnp.float32),
            pltpu.VMEM((H,), jnp.float32),
            pltpu.VMEM((H,), jnp.float32),
            pltpu.SemaphoreType.DMA,
            pltpu.SemaphoreType.DMA,
            pltpu.SemaphoreType.DMA,
            pltpu.SemaphoreType.DMA,
            pltpu.SemaphoreType.DMA,
            pltpu.SemaphoreType.DMA,
        ],
    )
    out2 = run(x1d, Ww2, Wp2, ln_scale, ln_bias)
    return out2.reshape(B, S, H)
